# baseline (device time: 25819 ns/iter reference)
import os

import jax
import jax.numpy as jnp
from jax import lax
from jax.experimental import pallas as pl
from jax.experimental.pallas import tpu as pltpu

try:
    _MODE = open(os.path.join(os.path.dirname(__file__),
                              "kernel_mode.txt")).read().strip()
except OSError:
    _MODE = "full"

NC = 2


def kernel(x, assign, W1, W2):
    T, D = x.shape
    E, _, F = W1.shape
    H = T // 2
    HC = H // NC
    assign2 = assign.reshape(T, 1)

    if _MODE in ("nop", "nopw"):
        def zbody(*refs):
            refs[-1][...] = jnp.zeros((T, D), jnp.float32)

        ops = (x, assign2) + (() if _MODE == "nop" else (W1, W2))
        return pl.pallas_call(
            zbody,
            out_shape=jax.ShapeDtypeStruct((T, D), jnp.float32),
            in_specs=[pl.BlockSpec(memory_space=pltpu.VMEM)] * len(ops),
            out_specs=pl.BlockSpec(memory_space=pltpu.VMEM),
        )(*ops)

    def body(x_ref, a_ref, w1_ref, w2_ref, out_ref,
             xsend, asend, xrecv, arecv, prsend, prrecv, fsend, frecv,
             xv, w1v, w2v, w1b, w2b, send_sems, recv_sems, wsems):
        my_x = lax.axis_index("x")
        my_y = lax.axis_index("y")
        xpeer = (1 - my_x, my_y)
        ypeer = (my_x, 1 - my_y)
        h0 = my_y * H

        cw1 = pltpu.make_async_copy(w1_ref, w1v, wsems.at[0])
        cw2 = pltpu.make_async_copy(w2_ref, w2v, wsems.at[1])
        cxv = pltpu.make_async_copy(x_ref.at[pl.ds(h0, H), :], xv, wsems.at[2])
        cw1.start()
        cw2.start()
        cxv.start()

        barrier_sem = pltpu.get_barrier_semaphore()
        for nbr in (xpeer, ypeer):
            pl.semaphore_signal(
                barrier_sem, inc=1, device_id=nbr,
                device_id_type=pl.DeviceIdType.MESH,
            )
        cxv.wait()
        xsend[...] = xv[...].astype(jnp.bfloat16)
        asend[...] = a_ref[pl.ds(h0, H), :]
        pl.semaphore_wait(barrier_sem, 2)

        cxs = []
        for c in range(NC):
            cx = pltpu.make_async_remote_copy(
                src_ref=xsend.at[pl.ds(c * HC, HC), :],
                dst_ref=xrecv.at[pl.ds(c * HC, HC), :],
                send_sem=send_sems.at[c], recv_sem=recv_sems.at[c],
                device_id=xpeer, device_id_type=pl.DeviceIdType.MESH,
            )
            cx.start()
            cxs.append(cx)
        ca = pltpu.make_async_remote_copy(
            src_ref=asend, dst_ref=arecv,
            send_sem=send_sems.at[NC], recv_sem=recv_sems.at[NC],
            device_id=xpeer, device_id_type=pl.DeviceIdType.MESH,
        )
        ca.start()

        cw1.wait()
        cw2.wait()
        w1b[...] = w1v[...].astype(jnp.bfloat16)
        w2b[...] = w2v[...].astype(jnp.bfloat16)

        def moe_local_experts(xb, ab):
            acc = jnp.zeros(xb.shape, jnp.float32)
            for e in range(E):
                eid = my_x * E + e
                xm = jnp.where(ab == eid, xb, jnp.bfloat16(0.0))
                h = jnp.dot(xm, w1b[e],
                            preferred_element_type=jnp.float32)
                h = jnp.maximum(h, 0.0).astype(jnp.bfloat16)
                acc = acc + jnp.dot(h, w2b[e],
                                    preferred_element_type=jnp.float32)
            return acc

        ca.wait()
        cps = []
        for c in range(NC):
            cxs[c].wait()
            prsend[pl.ds(c * HC, HC), :] = moe_local_experts(
                xrecv[pl.ds(c * HC, HC), :],
                arecv[pl.ds(c * HC, HC), :]).astype(jnp.bfloat16)
            cp = pltpu.make_async_remote_copy(
                src_ref=prsend.at[pl.ds(c * HC, HC), :],
                dst_ref=prrecv.at[pl.ds(c * HC, HC), :],
                send_sem=send_sems.at[NC + 1 + c],
                recv_sem=recv_sems.at[NC + 1 + c],
                device_id=xpeer, device_id_type=pl.DeviceIdType.MESH,
            )
            cp.start()
            cps.append(cp)

        p_own = moe_local_experts(xsend[...], asend[...])

        cfs = []
        for c in range(NC):
            cps[c].wait()
            fin = (p_own[c * HC:(c + 1) * HC, :]
                   + prrecv[pl.ds(c * HC, HC), :].astype(jnp.float32))
            fsend[pl.ds(c * HC, HC), :] = fin.astype(jnp.bfloat16)
            cf = pltpu.make_async_remote_copy(
                src_ref=fsend.at[pl.ds(c * HC, HC), :],
                dst_ref=frecv.at[pl.ds(c * HC, HC), :],
                send_sem=send_sems.at[2 * NC + 1 + c],
                recv_sem=recv_sems.at[2 * NC + 1 + c],
                device_id=ypeer, device_id_type=pl.DeviceIdType.MESH,
            )
            cf.start()
            cfs.append(cf)
            out_ref[pl.ds(h0 + c * HC, HC), :] = fin

        oh0 = (1 - my_y) * H
        for c in range(NC):
            cfs[c].wait()
            out_ref[pl.ds(oh0 + c * HC, HC), :] = (
                frecv[pl.ds(c * HC, HC), :].astype(jnp.float32))

    out_shape = jax.ShapeDtypeStruct((T, D), jnp.float32)
    return pl.pallas_call(
        body,
        out_shape=out_shape,
        in_specs=[
            pl.BlockSpec(memory_space=pltpu.MemorySpace.HBM),
            pl.BlockSpec(memory_space=pltpu.VMEM),
            pl.BlockSpec(memory_space=pltpu.MemorySpace.HBM),
            pl.BlockSpec(memory_space=pltpu.MemorySpace.HBM),
        ],
        out_specs=pl.BlockSpec(memory_space=pltpu.VMEM),
        scratch_shapes=[
            pltpu.VMEM((H, D), jnp.bfloat16),
            pltpu.VMEM((H, 1), jnp.int32),
            pltpu.VMEM((H, D), jnp.bfloat16),
            pltpu.VMEM((H, 1), jnp.int32),
            pltpu.VMEM((H, D), jnp.bfloat16),
            pltpu.VMEM((H, D), jnp.bfloat16),
            pltpu.VMEM((H, D), jnp.bfloat16),
            pltpu.VMEM((H, D), jnp.bfloat16),
            pltpu.VMEM((H, D), jnp.float32),
            pltpu.VMEM((E, D, F), jnp.float32),
            pltpu.VMEM((E, F, D), jnp.float32),
            pltpu.VMEM((E, D, F), jnp.bfloat16),
            pltpu.VMEM((E, F, D), jnp.bfloat16),
            pltpu.SemaphoreType.DMA((3 * NC + 1,)),
            pltpu.SemaphoreType.DMA((3 * NC + 1,)),
            pltpu.SemaphoreType.DMA((3,)),
        ],
        compiler_params=pltpu.CompilerParams(collective_id=0),
    )(x, assign2, W1, W2)
